# baseline (speedup 1.0000x reference)
import jax
import jax.numpy as jnp
from jax import lax
from jax.experimental import pallas as pl
from jax.experimental.pallas import tpu as pltpu

_B = 64
_WC = 256
_N = 257
_NPAD = 384
_R = 192


def _anchors_kernel(par_ref, out_ref):
    lane = lax.broadcasted_iota(jnp.int32, (1, _R), 1)
    row = jnp.zeros((1, _R), jnp.float32)
    for k in range(_B):
        row = jnp.where((lane >= 3 * k) & (lane < 3 * k + 3), par_ref[k], row)
    par = row.reshape(_R, 1)
    r = lax.broadcasted_iota(jnp.int32, (_R, 128), 0)
    l = lax.broadcasted_iota(jnp.int32, (_R, 128), 1)
    x = ((r % 3) * 128 + l).astype(jnp.float32)
    a = 0.25 / par ** 2
    s = jnp.sqrt(x ** 2 + a)
    integ_x = 0.5 * (x * s + a * jnp.log(jnp.abs(x + s)))
    s0 = jnp.sqrt(a)
    integ_0 = 0.5 * (a * jnp.log(jnp.abs(s0)))
    prev = 2.0 * par * (integ_x - integ_0)
    xs = prev + jnp.float32(_WC)
    xs = jnp.clip(xs - jnp.float32(_WC), 0.0, jnp.float32(_WC))
    out_ref[:, :] = jnp.round(xs).astype(jnp.int32)


def kernel(adv_patch, parabola_rate):
    del adv_patch
    out = pl.pallas_call(
        _anchors_kernel,
        in_specs=[pl.BlockSpec(memory_space=pltpu.SMEM)],
        out_shape=jax.ShapeDtypeStruct((_R, 128), jnp.int32),
    )(parabola_rate.reshape(_B))
    return out.reshape(_B, _NPAD, 1)[:, :_N, :]


# speedup vs baseline: 1.0216x; 1.0216x over previous
import jax
import jax.numpy as jnp
from jax import lax
from jax.experimental import pallas as pl

_B = 64
_W = 512
_WC = _W // 2
_N = _WC + 1
_NPAD = 384


def _anchors_kernel(par_ref, out_ref):
    par = par_ref[:].reshape(_B, 1)
    x = lax.broadcasted_iota(jnp.int32, (_B, _NPAD), 1).astype(jnp.float32)
    a = 0.25 / par ** 2
    s = jnp.sqrt(x ** 2 + a)
    integ_x = 0.5 * (x * s + a * jnp.log(jnp.abs(x + s)))
    s0 = jnp.sqrt(a)
    integ_0 = 0.5 * (a * jnp.log(jnp.abs(s0)))
    prev = 2.0 * par * (integ_x - integ_0)
    xs = prev + jnp.float32(_WC)
    xs = jnp.clip(xs - jnp.float32(_WC), 0.0, jnp.float32(_WC))
    val = jnp.round(xs).astype(jnp.int32)
    for b in range(_B):
        out_ref[pl.ds(b * _NPAD, _NPAD)] = val[b]


def kernel(adv_patch, parabola_rate):
    del adv_patch
    out = pl.pallas_call(
        _anchors_kernel,
        out_shape=jax.ShapeDtypeStruct((_B * _NPAD,), jnp.int32),
    )(parabola_rate.reshape(_B))
    return out.reshape(_B, _NPAD, 1)[:, :_N, :]
